# baseline (speedup 1.0000x reference)
import jax
import jax.numpy as jnp
from jax.experimental import pallas as pl
from jax.experimental.pallas import tpu as pltpu


_BS = 512


def _add_kernel(x_ref, e_ref, o_ref):
    o_ref[...] = x_ref[...] + e_ref[...][None, :, :]


def kernel(x, emb):
    B, S, D = x.shape
    grid = (S // _BS,)
    return pl.pallas_call(
        _add_kernel,
        grid=grid,
        in_specs=[
            pl.BlockSpec((B, _BS, D), lambda s: (0, s, 0)),
            pl.BlockSpec((_BS, D), lambda s: (s, 0)),
        ],
        out_specs=pl.BlockSpec((B, _BS, D), lambda s: (0, s, 0)),
        out_shape=jax.ShapeDtypeStruct(x.shape, x.dtype),
        compiler_params=pltpu.CompilerParams(
            dimension_semantics=("parallel",),
        ),
    )(x, emb[:S])
